# baseline (speedup 1.0000x reference)
import functools

import jax
import jax.numpy as jnp
from jax import lax
from jax.experimental import pallas as pl
from jax.experimental.pallas import tpu as pltpu
from jax.experimental.pallas import tpu_sc as plsc


def _make_sc_gather(b, input_dim, d, vocab):
    info = plsc.get_sparse_core_info()
    nc, ns = info.num_cores, info.num_subcores
    nw = nc * ns
    assert b % nw == 0
    rows_w = b // nw
    blk = 16
    nblk = rows_w // blk
    assert rows_w % blk == 0
    mlp_in = input_dim * d

    mesh = plsc.VectorSubcoreMesh(core_axis_name="c", subcore_axis_name="s")

    per_w = rows_w * input_dim
    nch = per_w // 16

    @functools.partial(
        pl.kernel,
        mesh=mesh,
        out_type=jax.ShapeDtypeStruct((b * input_dim, d), jnp.float32),
        scratch_types=[
            pltpu.VMEM((per_w,), jnp.int32),
            pltpu.VMEM((per_w,), jnp.int32),
            pltpu.VMEM((per_w,), jnp.float32),
            pltpu.VMEM((per_w, d), jnp.float32),
            pltpu.SemaphoreType.DMA,
        ],
        compiler_params=pltpu.CompilerParams(
            use_tc_tiling_on_sc=False, needs_layout_passes=False
        ),
    )
    def gather(table_hbm, idx_hbm, out_hbm, idx_v, widx_v, col_v, rows_v, sem):
        wid = lax.axis_index("s") * nc + lax.axis_index("c")
        base = wid * per_w
        pltpu.sync_copy(idx_hbm.at[pl.ds(base, per_w)], idx_v)

        def per_e(e, carry):
            off = e * vocab

            def mk_widx(k, carry2):
                widx_v[pl.ds(k * 16, 16)] = idx_v[pl.ds(k * 16, 16)] + off
                return carry2

            lax.fori_loop(0, nch, mk_widx, carry, unroll=False)
            pltpu.async_copy(table_hbm.at[widx_v], col_v, sem).wait()
            e_vec = jnp.full((16,), e, dtype=jnp.int32)
            lane = lax.iota(jnp.int32, 16)

            def xpose(k, carry2):
                r_vec = lane + k * 16
                plsc.store_scatter(
                    rows_v, [r_vec, e_vec], col_v[pl.ds(k * 16, 16)]
                )
                return carry2

            lax.fori_loop(0, nch, xpose, carry, unroll=False)
            return carry

        lax.fori_loop(0, d, per_e, 0, unroll=False)
        pltpu.sync_copy(rows_v, out_hbm.at[pl.ds(base, per_w)])

    return gather


def _mlp_body(flat_ref, w1_ref, b1_ref, w2_ref, b2_ref, out_ref):
    h = jnp.dot(flat_ref[...], w1_ref[...], preferred_element_type=jnp.float32)
    h = jnp.maximum(h + b1_ref[...], 0.0)
    out_ref[...] = (
        jnp.dot(h, w2_ref[...], preferred_element_type=jnp.float32) + b2_ref[...]
    )


def _mlp(flat, W1, b1, W2, b2, blk):
    b, mlp_in = flat.shape
    hidden = W1.shape[1]
    out_sz = W2.shape[1]
    return pl.pallas_call(
        _mlp_body,
        grid=(b // blk,),
        in_specs=[
            pl.BlockSpec((blk, mlp_in), lambda i: (i, 0)),
            pl.BlockSpec((mlp_in, hidden), lambda i: (0, 0)),
            pl.BlockSpec((1, hidden), lambda i: (0, 0)),
            pl.BlockSpec((hidden, out_sz), lambda i: (0, 0)),
            pl.BlockSpec((1, out_sz), lambda i: (0, 0)),
        ],
        out_specs=pl.BlockSpec((blk, out_sz), lambda i: (i, 0)),
        out_shape=jax.ShapeDtypeStruct((b, out_sz), jnp.float32),
    )(flat, W1, b1.reshape(1, -1), W2, b2.reshape(1, -1))


def kernel(x, emb_table, W1, b1, W2, b2):
    b, input_dim = x.shape
    vocab, d = emb_table.shape
    gather = _make_sc_gather(b, input_dim, d, vocab)
    table_flat = emb_table.T.reshape(vocab * d)
    flat = gather(table_flat, x.reshape(b * input_dim)).reshape(b, input_dim * d)
    return _mlp(flat, W1, b1, W2, b2, blk=512)


# speedup vs baseline: 4.7499x; 4.7499x over previous
import functools

import jax
import jax.numpy as jnp
from jax import lax
from jax.experimental import pallas as pl
from jax.experimental.pallas import tpu as pltpu
from jax.experimental.pallas import tpu_sc as plsc


def _make_sc_gather(b, input_dim, d, vocab):
    info = plsc.get_sparse_core_info()
    nc, ns = info.num_cores, info.num_subcores
    nw = nc * ns
    assert b % nw == 0
    rows_w = b // nw
    blk = 16
    nblk = rows_w // blk
    assert rows_w % blk == 0
    mlp_in = input_dim * d

    mesh = plsc.VectorSubcoreMesh(core_axis_name="c", subcore_axis_name="s")

    per_w = rows_w * input_dim
    chunk = 416
    nchunk = per_w // chunk
    assert per_w % chunk == 0

    @functools.partial(
        pl.kernel,
        mesh=mesh,
        out_type=jax.ShapeDtypeStruct((b * input_dim, d), jnp.float32),
        scratch_types=[
            pltpu.VMEM((per_w,), jnp.int32),
            pltpu.VMEM((chunk, 128), jnp.float32),
            pltpu.VMEM((chunk, d), jnp.float32),
            pltpu.SemaphoreType.DMA,
        ],
        compiler_params=pltpu.CompilerParams(use_tc_tiling_on_sc=False),
    )
    def gather(table_hbm, idx_hbm, out_hbm, idx_v, wide_v, comp_v, sem):
        wid = lax.axis_index("s") * nc + lax.axis_index("c")
        base = wid * per_w
        pltpu.sync_copy(idx_hbm.at[pl.ds(base, per_w)], idx_v)

        def do_chunk(ci, carry):
            off = ci * chunk
            pltpu.async_copy(
                table_hbm.at[idx_v.at[pl.ds(off, chunk)]], wide_v, sem
            ).wait()

            def comp_row(r, carry2):
                for h in range(0, d, 16):
                    comp_v[r, pl.ds(h, 16)] = wide_v[r, pl.ds(h, 16)]
                return carry2

            lax.fori_loop(0, chunk, comp_row, carry, unroll=False)
            pltpu.sync_copy(comp_v, out_hbm.at[pl.ds(base + off, chunk)])
            return carry

        lax.fori_loop(0, nchunk, do_chunk, 0, unroll=False)

    return gather


def _mlp_body(flat_ref, w1_ref, b1_ref, w2_ref, b2_ref, out_ref):
    h = jnp.dot(flat_ref[...], w1_ref[...], preferred_element_type=jnp.float32)
    h = jnp.maximum(h + b1_ref[...], 0.0)
    out_ref[...] = (
        jnp.dot(h, w2_ref[...], preferred_element_type=jnp.float32) + b2_ref[...]
    )


def _mlp(flat, W1, b1, W2, b2, blk):
    b, mlp_in = flat.shape
    hidden = W1.shape[1]
    out_sz = W2.shape[1]
    return pl.pallas_call(
        _mlp_body,
        grid=(b // blk,),
        in_specs=[
            pl.BlockSpec((blk, mlp_in), lambda i: (i, 0)),
            pl.BlockSpec((mlp_in, hidden), lambda i: (0, 0)),
            pl.BlockSpec((1, hidden), lambda i: (0, 0)),
            pl.BlockSpec((hidden, out_sz), lambda i: (0, 0)),
            pl.BlockSpec((1, out_sz), lambda i: (0, 0)),
        ],
        out_specs=pl.BlockSpec((blk, out_sz), lambda i: (i, 0)),
        out_shape=jax.ShapeDtypeStruct((b, out_sz), jnp.float32),
    )(flat, W1, b1.reshape(1, -1), W2, b2.reshape(1, -1))


def kernel(x, emb_table, W1, b1, W2, b2):
    b, input_dim = x.shape
    vocab, d = emb_table.shape
    gather = _make_sc_gather(b, input_dim, d, vocab)
    table_pad = jnp.pad(emb_table, ((0, 0), (0, 128 - d)))
    flat = gather(table_pad, x.reshape(b * input_dim)).reshape(b, input_dim * d)
    return _mlp(flat, W1, b1, W2, b2, blk=512)


# speedup vs baseline: 5.0341x; 1.0598x over previous
import functools

import jax
import jax.numpy as jnp
from jax import lax
from jax.experimental import pallas as pl
from jax.experimental.pallas import tpu as pltpu
from jax.experimental.pallas import tpu_sc as plsc


def _make_sc_gather(b, input_dim, d, vocab):
    info = plsc.get_sparse_core_info()
    nc, ns = info.num_cores, info.num_subcores
    nw = nc * ns
    assert b % nw == 0
    rows_w = b // nw
    blk = 16
    nblk = rows_w // blk
    assert rows_w % blk == 0
    mlp_in = input_dim * d

    mesh = plsc.VectorSubcoreMesh(core_axis_name="c", subcore_axis_name="s")

    per_w = rows_w * input_dim

    @functools.partial(
        pl.kernel,
        mesh=mesh,
        out_type=jax.ShapeDtypeStruct((b * input_dim, d), jnp.float32),
        scratch_types=[
            pltpu.VMEM((per_w,), jnp.int32),
            pltpu.VMEM((per_w, d), jnp.float32),
            pltpu.SemaphoreType.DMA,
        ],
        compiler_params=pltpu.CompilerParams(use_tc_tiling_on_sc=False),
    )
    def gather(table_hbm, idx_hbm, out_hbm, idx_v, rows_v, sem):
        wid = lax.axis_index("s") * nc + lax.axis_index("c")
        base = wid * per_w
        pltpu.sync_copy(idx_hbm.at[pl.ds(base, per_w)], idx_v)
        pltpu.async_copy(table_hbm.at[idx_v], rows_v, sem).wait()
        pltpu.sync_copy(rows_v, out_hbm.at[pl.ds(base, per_w)])

    return gather


def _mlp_body(flat_ref, w1_ref, b1_ref, w2_ref, b2_ref, out_ref):
    h = jnp.dot(flat_ref[...], w1_ref[...], preferred_element_type=jnp.float32)
    h = jnp.maximum(h + b1_ref[...], 0.0)
    out_ref[...] = (
        jnp.dot(h, w2_ref[...], preferred_element_type=jnp.float32) + b2_ref[...]
    )


def _mlp(flat, W1, b1, W2, b2, blk):
    b, mlp_in = flat.shape
    hidden = W1.shape[1]
    out_sz = W2.shape[1]
    return pl.pallas_call(
        _mlp_body,
        grid=(b // blk,),
        in_specs=[
            pl.BlockSpec((blk, mlp_in), lambda i: (i, 0)),
            pl.BlockSpec((mlp_in, hidden), lambda i: (0, 0)),
            pl.BlockSpec((1, hidden), lambda i: (0, 0)),
            pl.BlockSpec((hidden, out_sz), lambda i: (0, 0)),
            pl.BlockSpec((1, out_sz), lambda i: (0, 0)),
        ],
        out_specs=pl.BlockSpec((blk, out_sz), lambda i: (i, 0)),
        out_shape=jax.ShapeDtypeStruct((b, out_sz), jnp.float32),
    )(flat, W1, b1.reshape(1, -1), W2, b2.reshape(1, -1))


def kernel(x, emb_table, W1, b1, W2, b2):
    b, input_dim = x.shape
    vocab, d = emb_table.shape
    gather = _make_sc_gather(b, input_dim, d, vocab)
    flat = gather(emb_table, x.reshape(b * input_dim)).reshape(b, input_dim * d)
    return _mlp(flat, W1, b1, W2, b2, blk=512)


# speedup vs baseline: 5.0404x; 1.0013x over previous
import functools

import jax
import jax.numpy as jnp
from jax import lax
from jax.experimental import pallas as pl
from jax.experimental.pallas import tpu as pltpu
from jax.experimental.pallas import tpu_sc as plsc


def _make_sc_gather(b, input_dim, d):
    info = plsc.get_sparse_core_info()
    nc, ns = info.num_cores, info.num_subcores
    nw = nc * ns
    assert b % nw == 0
    per_w = (b // nw) * input_dim
    assert per_w % 8 == 0

    mesh = plsc.VectorSubcoreMesh(core_axis_name="c", subcore_axis_name="s")

    @functools.partial(
        pl.kernel,
        mesh=mesh,
        out_type=jax.ShapeDtypeStruct((b * input_dim, d), jnp.float32),
        scratch_types=[
            pltpu.VMEM((per_w,), jnp.int32),
            pltpu.VMEM((per_w, d), jnp.float32),
            pltpu.SemaphoreType.DMA,
        ],
        compiler_params=pltpu.CompilerParams(use_tc_tiling_on_sc=False),
    )
    def gather(table_hbm, idx_hbm, out_hbm, idx_v, rows_v, sem):
        wid = lax.axis_index("s") * nc + lax.axis_index("c")
        base = wid * per_w
        pltpu.sync_copy(idx_hbm.at[pl.ds(base, per_w)], idx_v)
        pltpu.async_copy(table_hbm.at[idx_v], rows_v, sem).wait()
        pltpu.sync_copy(rows_v, out_hbm.at[pl.ds(base, per_w)])

    return gather


def _mlp_body(flat_ref, w1_ref, b1_ref, w2_ref, b2_ref, out_ref):
    h = jnp.dot(flat_ref[...], w1_ref[...], preferred_element_type=jnp.float32)
    h = jnp.maximum(h + b1_ref[...], 0.0)
    out_ref[...] = (
        jnp.dot(h, w2_ref[...], preferred_element_type=jnp.float32) + b2_ref[...]
    )


def _mlp(flat, W1, b1, W2, b2, blk):
    b, mlp_in = flat.shape
    hidden = W1.shape[1]
    out_sz = W2.shape[1]
    return pl.pallas_call(
        _mlp_body,
        grid=(b // blk,),
        in_specs=[
            pl.BlockSpec((blk, mlp_in), lambda i: (i, 0)),
            pl.BlockSpec((mlp_in, hidden), lambda i: (0, 0)),
            pl.BlockSpec((1, hidden), lambda i: (0, 0)),
            pl.BlockSpec((hidden, out_sz), lambda i: (0, 0)),
            pl.BlockSpec((1, out_sz), lambda i: (0, 0)),
        ],
        out_specs=pl.BlockSpec((blk, out_sz), lambda i: (i, 0)),
        out_shape=jax.ShapeDtypeStruct((b, out_sz), jnp.float32),
    )(flat, W1, b1.reshape(1, -1), W2, b2.reshape(1, -1))


def kernel(x, emb_table, W1, b1, W2, b2):
    b, input_dim = x.shape
    vocab, d = emb_table.shape
    gather = _make_sc_gather(b, input_dim, d)
    flat = gather(emb_table, x.reshape(b * input_dim)).reshape(b, input_dim * d)
    return _mlp(flat, W1, b1, W2, b2, blk=512)
